# baseline (speedup 1.0000x reference)
import functools

import jax
import jax.numpy as jnp
from jax.experimental import pallas as pl
from jax.experimental.pallas import tpu as pltpu

_S = 2048


def _body(xb_ref, halo_ref, cls_ref, pos_ref, out_ref, *, s):
    k = pl.program_id(0)
    first = jnp.where(k == 0, cls_ref[0], halo_ref[0, 0])
    xblk = xb_ref[0]
    rolled = pltpu.roll(xblk, shift=1, axis=0)
    row_ids = jax.lax.broadcasted_iota(jnp.int32, xblk.shape, 0)
    shifted = jnp.where(row_ids == 0, first, rolled)
    out_ref[0] = shifted + pos_ref[...]


def kernel(x, cls_token, pos_table):
    batch, seq_len, d = x.shape
    s = _S
    kx = seq_len // s
    grid_k = kx + 1
    halo = x[:, s - 1 :: s, :].reshape(batch, kx, 1, d)

    def xb_index(k, b):
        return (jnp.where(k == kx, batch - 1, b), jnp.minimum(k, kx - 1), 0)

    def halo_index(k, b):
        return (b, jnp.maximum(k - 1, 0), 0, 0)

    out = pl.pallas_call(
        functools.partial(_body, s=s),
        grid=(grid_k, batch),
        in_specs=[
            pl.BlockSpec((1, s, d), xb_index),
            pl.BlockSpec((1, 1, 1, d), halo_index),
            pl.BlockSpec((1, 1, d), lambda k, b: (0, 0, 0)),
            pl.BlockSpec((s, d), lambda k, b: (k, 0)),
        ],
        out_specs=pl.BlockSpec((1, s, d), lambda k, b: (b, k, 0)),
        out_shape=jax.ShapeDtypeStruct((batch, seq_len + 1, d), x.dtype),
        compiler_params=pltpu.CompilerParams(
            dimension_semantics=("arbitrary", "arbitrary"),
        ),
    )(x, halo, cls_token, pos_table)
    return out
